# Initial kernel scaffold; baseline (speedup 1.0000x reference)
import functools

import jax
import jax.numpy as jnp
from jax import lax
from jax.experimental import pallas as pl
from jax.experimental.pallas import tpu as pltpu
from jax.experimental.pallas import tpu_sc as plsc

B0, B1 = 16384, 200
DIM = 32
NUM_TOK = B0 * B1
NC, NS = 2, 16
NW = NC * NS
W = NUM_TOK // NW
C = 128
NB = 8
M = W // C
BLOCKS = M // NB

_mesh = plsc.VectorSubcoreMesh(core_axis_name="c", subcore_axis_name="s")


@functools.partial(
    pl.kernel,
    mesh=_mesh,
    out_type=jax.ShapeDtypeStruct((NUM_TOK, DIM), jnp.float32),
    scratch_types=[
        pltpu.VMEM((2, NB, C), jnp.int32),
        pltpu.VMEM((2, NB * C, DIM), jnp.float32),
        pltpu.SemaphoreType.DMA,
        pltpu.SemaphoreType.DMA,
        pltpu.SemaphoreType.DMA,
    ],
)
def _emb_lookup(idx_hbm, table_hbm, out_hbm, idx_v, rows_v, sem_i, sem_g, sem_o):
    wid = lax.axis_index("s") * NC + lax.axis_index("c")

    pltpu.async_copy(idx_hbm.at[wid, pl.ds(0, NB)], idx_v.at[0], sem_i).wait()

    def block(b, carry):
        slot = lax.rem(b, 2)
        nslot = 1 - slot

        @pl.when(b >= 2)
        def _drain_prev_writeback():
            pltpu.make_async_copy(
                rows_v.at[slot],
                out_hbm.at[pl.ds(0, NB * C)],
                sem_o,
            ).wait()

        @pl.when(b >= 1)
        def _wait_idx():
            pltpu.make_async_copy(
                idx_hbm.at[wid, pl.ds(0, NB)], idx_v.at[slot], sem_i
            ).wait()

        @pl.when(b + 1 < BLOCKS)
        def _prefetch_idx():
            pltpu.async_copy(
                idx_hbm.at[wid, pl.ds((b + 1) * NB, NB)], idx_v.at[nslot], sem_i
            )

        gathers = []
        for j in range(NB):
            gathers.append(
                pltpu.async_copy(
                    table_hbm.at[idx_v.at[slot, j]],
                    rows_v.at[slot, pl.ds(j * C, C)],
                    sem_g,
                )
            )
        for g in gathers:
            g.wait()

        base = wid * W + b * (NB * C)
        pltpu.async_copy(rows_v.at[slot], out_hbm.at[pl.ds(base, NB * C)], sem_o)
        return carry

    lax.fori_loop(0, BLOCKS, block, 0)

    for _ in range(2):
        pltpu.make_async_copy(
            rows_v.at[0], out_hbm.at[pl.ds(0, NB * C)], sem_o
        ).wait()


def kernel(token_ids, weight):
    idx = jnp.asarray(token_ids, jnp.int32).reshape(NW, M, C)
    out = _emb_lookup(idx, weight)
    return out.reshape(B0, B1, DIM)


# speedup vs baseline: 5.0335x; 5.0335x over previous
import functools

import jax
import jax.numpy as jnp
from jax import lax
from jax.experimental import pallas as pl
from jax.experimental.pallas import tpu as pltpu
from jax.experimental.pallas import tpu_sc as plsc

B0, B1 = 16384, 200
DIM = 32
NUM_TOK = B0 * B1
NC, NS = 2, 16
NW = NC * NS
W = NUM_TOK // NW
C = 128
NB = 8
M = W // C
BLOCKS = M // NB

_mesh = plsc.VectorSubcoreMesh(core_axis_name="c", subcore_axis_name="s")


@functools.partial(
    pl.kernel,
    mesh=_mesh,
    compiler_params=pltpu.CompilerParams(use_tc_tiling_on_sc=False),
    out_type=jax.ShapeDtypeStruct((NUM_TOK, DIM), jnp.float32),
    scratch_types=[
        pltpu.VMEM((2, NB, C), jnp.int32),
        pltpu.VMEM((2, NB * C, DIM), jnp.float32),
        pltpu.SemaphoreType.DMA,
        pltpu.SemaphoreType.DMA,
        pltpu.SemaphoreType.DMA,
    ],
)
def _emb_lookup(idx_hbm, table_hbm, out_hbm, idx_v, rows_v, sem_i, sem_g, sem_o):
    wid = lax.axis_index("s") * NC + lax.axis_index("c")

    pltpu.async_copy(idx_hbm.at[wid, pl.ds(0, NB)], idx_v.at[0], sem_i).wait()

    def block(b, carry):
        slot = lax.rem(b, 2)
        nslot = 1 - slot

        @pl.when(b >= 2)
        def _drain_prev_writeback():
            pltpu.make_async_copy(
                rows_v.at[slot],
                out_hbm.at[pl.ds(0, NB * C)],
                sem_o,
            ).wait()

        @pl.when(b >= 1)
        def _wait_idx():
            pltpu.make_async_copy(
                idx_hbm.at[wid, pl.ds(0, NB)], idx_v.at[slot], sem_i
            ).wait()

        @pl.when(b + 1 < BLOCKS)
        def _prefetch_idx():
            pltpu.async_copy(
                idx_hbm.at[wid, pl.ds((b + 1) * NB, NB)], idx_v.at[nslot], sem_i
            )

        gathers = []
        for j in range(NB):
            gathers.append(
                pltpu.async_copy(
                    table_hbm.at[idx_v.at[slot, j]],
                    rows_v.at[slot, pl.ds(j * C, C)],
                    sem_g,
                )
            )
        for g in gathers:
            g.wait()

        base = wid * W + b * (NB * C)
        pltpu.async_copy(rows_v.at[slot], out_hbm.at[pl.ds(base, NB * C)], sem_o)
        return carry

    lax.fori_loop(0, BLOCKS, block, 0)

    for _ in range(2):
        pltpu.make_async_copy(
            rows_v.at[0], out_hbm.at[pl.ds(0, NB * C)], sem_o
        ).wait()


def kernel(token_ids, weight):
    idx = jnp.asarray(token_ids, jnp.int32).reshape(NW, M, C)
    out = _emb_lookup(idx, weight)
    return out.reshape(B0, B1, DIM)
